# baseline (speedup 1.0000x reference)
import functools

import jax
import jax.numpy as jnp
from jax import lax
from jax.experimental import pallas as pl
from jax.experimental.pallas import tpu as pltpu
from jax.experimental.pallas import tpu_sc as plsc

NUM_FEATS = 26
DIM = 32
BATCH = 16384
NC = 2
NS = 16
NW = NC * NS
BPW = BATCH // NW


def _cat_embed(*args):
    mesh = plsc.VectorSubcoreMesh(core_axis_name="c", subcore_axis_name="s")

    @functools.partial(
        pl.kernel,
        mesh=mesh,
        out_type=jax.ShapeDtypeStruct((NUM_FEATS, BATCH, DIM), jnp.float32),
        scratch_types=[
            pltpu.VMEM((BPW,), jnp.int32),
            pltpu.VMEM((BPW, DIM), jnp.float32),
            pltpu.SemaphoreType.DMA,
        ],
        compiler_params=pltpu.CompilerParams(use_tc_tiling_on_sc=False),
    )
    def k(*refs):
        idx_refs = refs[:NUM_FEATS]
        tbl_refs = refs[NUM_FEATS:2 * NUM_FEATS]
        out = refs[2 * NUM_FEATS]
        idx_v, rows_v, sem = refs[2 * NUM_FEATS + 1:]

        wid = lax.axis_index("s") * NC + lax.axis_index("c")
        base = wid * BPW
        for f in range(NUM_FEATS):
            pltpu.sync_copy(idx_refs[f].at[pl.ds(base, BPW)], idx_v)
            pltpu.async_copy(tbl_refs[f].at[idx_v], rows_v, sem).wait()
            pltpu.sync_copy(rows_v, out.at[f, pl.ds(base, BPW), :])

    out3 = k(*args)
    return out3.transpose(1, 0, 2).reshape(BATCH, NUM_FEATS * DIM)


def kernel(f00, f01, f02, f03, f04, f05, f06, f07, f08, f09, f10, f11, f12,
           f13, f14, f15, f16, f17, f18, f19, f20, f21, f22, f23, f24, f25,
           W_f00, W_f01, W_f02, W_f03, W_f04, W_f05, W_f06, W_f07, W_f08,
           W_f09, W_f10, W_f11, W_f12, W_f13, W_f14, W_f15, W_f16, W_f17,
           W_f18, W_f19, W_f20, W_f21, W_f22, W_f23, W_f24, W_f25):
    return _cat_embed(
        f00, f01, f02, f03, f04, f05, f06, f07, f08, f09, f10, f11, f12,
        f13, f14, f15, f16, f17, f18, f19, f20, f21, f22, f23, f24, f25,
        W_f00, W_f01, W_f02, W_f03, W_f04, W_f05, W_f06, W_f07, W_f08,
        W_f09, W_f10, W_f11, W_f12, W_f13, W_f14, W_f15, W_f16, W_f17,
        W_f18, W_f19, W_f20, W_f21, W_f22, W_f23, W_f24, W_f25)


# speedup vs baseline: 1.0746x; 1.0746x over previous
import functools

import jax
import jax.numpy as jnp
from jax import lax
from jax.experimental import pallas as pl
from jax.experimental.pallas import tpu as pltpu
from jax.experimental.pallas import tpu_sc as plsc

NUM_FEATS = 26
DIM = 32
BATCH = 16384
NC = 2
NS = 16
NW = NC * NS
BPW = BATCH // NW


def _cat_embed(*args):
    mesh = plsc.VectorSubcoreMesh(core_axis_name="c", subcore_axis_name="s")

    @functools.partial(
        pl.kernel,
        mesh=mesh,
        out_type=jax.ShapeDtypeStruct((BATCH, NUM_FEATS * DIM), jnp.float32),
        scratch_types=[
            pltpu.VMEM((BPW,), jnp.int32),
            pltpu.VMEM((BPW, DIM), jnp.float32),
            pltpu.SemaphoreType.DMA,
        ],
        compiler_params=pltpu.CompilerParams(use_tc_tiling_on_sc=False),
    )
    def k(*refs):
        idx_refs = refs[:NUM_FEATS]
        tbl_refs = refs[NUM_FEATS:2 * NUM_FEATS]
        out = refs[2 * NUM_FEATS]
        idx_v, rows_v, sem = refs[2 * NUM_FEATS + 1:]

        wid = lax.axis_index("s") * NC + lax.axis_index("c")
        base = wid * BPW
        for f in range(NUM_FEATS):
            pltpu.sync_copy(idx_refs[f].at[pl.ds(base, BPW)], idx_v)
            pltpu.async_copy(tbl_refs[f].at[idx_v], rows_v, sem).wait()
            pltpu.sync_copy(
                rows_v, out.at[pl.ds(base, BPW), pl.ds(f * DIM, DIM)]
            )

    return k(*args)


def kernel(f00, f01, f02, f03, f04, f05, f06, f07, f08, f09, f10, f11, f12,
           f13, f14, f15, f16, f17, f18, f19, f20, f21, f22, f23, f24, f25,
           W_f00, W_f01, W_f02, W_f03, W_f04, W_f05, W_f06, W_f07, W_f08,
           W_f09, W_f10, W_f11, W_f12, W_f13, W_f14, W_f15, W_f16, W_f17,
           W_f18, W_f19, W_f20, W_f21, W_f22, W_f23, W_f24, W_f25):
    return _cat_embed(
        f00, f01, f02, f03, f04, f05, f06, f07, f08, f09, f10, f11, f12,
        f13, f14, f15, f16, f17, f18, f19, f20, f21, f22, f23, f24, f25,
        W_f00, W_f01, W_f02, W_f03, W_f04, W_f05, W_f06, W_f07, W_f08,
        W_f09, W_f10, W_f11, W_f12, W_f13, W_f14, W_f15, W_f16, W_f17,
        W_f18, W_f19, W_f20, W_f21, W_f22, W_f23, W_f24, W_f25)


# speedup vs baseline: 1.0759x; 1.0013x over previous
import functools

import jax
import jax.numpy as jnp
from jax import lax
from jax.experimental import pallas as pl
from jax.experimental.pallas import tpu as pltpu
from jax.experimental.pallas import tpu_sc as plsc

NUM_FEATS = 26
DIM = 32
CARD = 100000
BATCH = 16384
NC = 2
NS = 16
NW = NC * NS
BPW = BATCH // NW


def _relayout(w):
    z = jax.lax.optimization_barrier(w.reshape(CARD // 4, 4 * DIM))
    return z.reshape(CARD, DIM)


def _cat_embed(*args):
    mesh = plsc.VectorSubcoreMesh(core_axis_name="c", subcore_axis_name="s")

    @functools.partial(
        pl.kernel,
        mesh=mesh,
        out_type=jax.ShapeDtypeStruct((BATCH, NUM_FEATS * DIM), jnp.float32),
        scratch_types=[
            pltpu.VMEM((BPW,), jnp.int32),
            pltpu.VMEM((BPW, DIM), jnp.float32),
            pltpu.SemaphoreType.DMA,
        ],
        compiler_params=pltpu.CompilerParams(use_tc_tiling_on_sc=False),
    )
    def k(*refs):
        idx_refs = refs[:NUM_FEATS]
        tbl_refs = refs[NUM_FEATS:2 * NUM_FEATS]
        out = refs[2 * NUM_FEATS]
        idx_v, rows_v, sem = refs[2 * NUM_FEATS + 1:]

        wid = lax.axis_index("s") * NC + lax.axis_index("c")
        base = wid * BPW
        for f in range(NUM_FEATS):
            pltpu.sync_copy(idx_refs[f].at[pl.ds(base, BPW)], idx_v)
            pltpu.async_copy(tbl_refs[f].at[idx_v], rows_v, sem).wait()
            pltpu.sync_copy(
                rows_v, out.at[pl.ds(base, BPW), pl.ds(f * DIM, DIM)]
            )

    idx_args = args[:NUM_FEATS]
    tbl_args = [_relayout(w) for w in args[NUM_FEATS:]]
    return k(*idx_args, *tbl_args)


def kernel(f00, f01, f02, f03, f04, f05, f06, f07, f08, f09, f10, f11, f12,
           f13, f14, f15, f16, f17, f18, f19, f20, f21, f22, f23, f24, f25,
           W_f00, W_f01, W_f02, W_f03, W_f04, W_f05, W_f06, W_f07, W_f08,
           W_f09, W_f10, W_f11, W_f12, W_f13, W_f14, W_f15, W_f16, W_f17,
           W_f18, W_f19, W_f20, W_f21, W_f22, W_f23, W_f24, W_f25):
    return _cat_embed(
        f00, f01, f02, f03, f04, f05, f06, f07, f08, f09, f10, f11, f12,
        f13, f14, f15, f16, f17, f18, f19, f20, f21, f22, f23, f24, f25,
        W_f00, W_f01, W_f02, W_f03, W_f04, W_f05, W_f06, W_f07, W_f08,
        W_f09, W_f10, W_f11, W_f12, W_f13, W_f14, W_f15, W_f16, W_f17,
        W_f18, W_f19, W_f20, W_f21, W_f22, W_f23, W_f24, W_f25)


# speedup vs baseline: 1.0760x; 1.0001x over previous
import functools

import jax
import jax.numpy as jnp
from jax import lax
from jax.experimental import pallas as pl
from jax.experimental.pallas import tpu as pltpu
from jax.experimental.pallas import tpu_sc as plsc

NUM_FEATS = 26
DIM = 32
BATCH = 16384
NC = 2
NS = 16
NW = NC * NS
BPW = BATCH // NW


def _cat_embed(*args):
    mesh = plsc.VectorSubcoreMesh(core_axis_name="c", subcore_axis_name="s")

    @functools.partial(
        pl.kernel,
        mesh=mesh,
        out_type=jax.ShapeDtypeStruct((BATCH, NUM_FEATS * DIM), jnp.float32),
        scratch_types=[
            pltpu.VMEM((BPW,), jnp.int32),
            pltpu.VMEM((BPW, DIM), jnp.float32),
            pltpu.SemaphoreType.DMA,
        ],
        compiler_params=pltpu.CompilerParams(use_tc_tiling_on_sc=False),
    )
    def k(*refs):
        idx_refs = refs[:NUM_FEATS]
        tbl_refs = refs[NUM_FEATS:2 * NUM_FEATS]
        out = refs[2 * NUM_FEATS]
        idx_v, rows_v, sem = refs[2 * NUM_FEATS + 1:]

        wid = lax.axis_index("s") * NC + lax.axis_index("c")
        base = wid * BPW
        for f in range(NUM_FEATS):
            pltpu.sync_copy(idx_refs[f].at[pl.ds(base, BPW)], idx_v)
            pltpu.async_copy(tbl_refs[f].at[idx_v], rows_v, sem).wait()
            pltpu.sync_copy(
                rows_v, out.at[pl.ds(base, BPW), pl.ds(f * DIM, DIM)]
            )

    return k(*args)


def kernel(f00, f01, f02, f03, f04, f05, f06, f07, f08, f09, f10, f11, f12,
           f13, f14, f15, f16, f17, f18, f19, f20, f21, f22, f23, f24, f25,
           W_f00, W_f01, W_f02, W_f03, W_f04, W_f05, W_f06, W_f07, W_f08,
           W_f09, W_f10, W_f11, W_f12, W_f13, W_f14, W_f15, W_f16, W_f17,
           W_f18, W_f19, W_f20, W_f21, W_f22, W_f23, W_f24, W_f25):
    return _cat_embed(
        f00, f01, f02, f03, f04, f05, f06, f07, f08, f09, f10, f11, f12,
        f13, f14, f15, f16, f17, f18, f19, f20, f21, f22, f23, f24, f25,
        W_f00, W_f01, W_f02, W_f03, W_f04, W_f05, W_f06, W_f07, W_f08,
        W_f09, W_f10, W_f11, W_f12, W_f13, W_f14, W_f15, W_f16, W_f17,
        W_f18, W_f19, W_f20, W_f21, W_f22, W_f23, W_f24, W_f25)
